# baseline (speedup 1.0000x reference)
import jax
import jax.numpy as jnp
from jax import lax
from jax.experimental import pallas as pl
from jax.experimental.pallas import tpu as pltpu
import jax.experimental.pallas.tpu_sc as plsc

N_NODES = 10000
N_EDGES = 320000
HIDDEN = 128

NC = 2
NS = 16
CH = 1000
N_CH = N_NODES // CH
DEG_W = 16

B = 80
NB = N_EDGES // NC // NS // B
NBUF = 3
NB_MAIN = (NB // NBUF) * NBUF

_mesh = plsc.VectorSubcoreMesh(core_axis_name="c", subcore_axis_name="s")
_sc_params = pltpu.CompilerParams(use_tc_tiling_on_sc=False)


FIRE = 25


def _deg_body(dst_hbm, ones_hbm, zeros_hbm, deg_out, idx_v, ones_v, sem, deg_sh):
    cid = lax.axis_index("c")
    sid = lax.axis_index("s")

    pltpu.sync_copy(ones_hbm, ones_v)
    pltpu.sync_copy(dst_hbm.at[cid, sid], idx_v)

    @pl.when(sid < N_CH)
    def _zero_chunk():
        pltpu.sync_copy(zeros_hbm, deg_sh.at[pl.ds(sid * CH, CH)])

    plsc.subcore_barrier()

    def chunk(c, carry):
        def fire(j, carry2):
            pltpu.async_copy(ones_v, deg_sh.at[idx_v.at[c * FIRE + j]], sem,
                             add=True)
            return carry2

        lax.fori_loop(0, FIRE, fire, 0)

        def drain(j, carry2):
            pltpu.make_async_copy(ones_v, deg_sh.at[idx_v.at[0]], sem).wait()
            return carry2

        lax.fori_loop(0, FIRE, drain, 0)
        return carry

    lax.fori_loop(0, NB // FIRE, chunk, 0)
    plsc.subcore_barrier()

    @pl.when(sid < N_CH)
    def _write_chunk():
        pltpu.sync_copy(
            deg_sh.at[pl.ds(sid * CH, CH)],
            deg_out.at[cid, pl.ds(sid * CH, CH)],
        )


_deg_kernel = pl.kernel(
    _deg_body,
    out_type=jax.ShapeDtypeStruct((NC, N_NODES, DEG_W), jnp.float32),
    mesh=_mesh,
    compiler_params=_sc_params,
    scratch_types=[
        pltpu.VMEM((NB, B), jnp.int32),
        pltpu.VMEM((B, DEG_W), jnp.float32),
        pltpu.SemaphoreType.DMA,
        pltpu.VMEM_SHARED((N_NODES, DEG_W), jnp.float32),
    ],
)


def _agg_body(zs_hbm, src_hbm, dst_hbm, acc_out,
              sidx_v, didx_v, rows_refs, sem_refs, acc_sh):
    cid = lax.axis_index("c")
    sid = lax.axis_index("s")
    slots = tuple(zip(rows_refs, sem_refs))

    pltpu.sync_copy(src_hbm.at[cid, sid], sidx_v)
    pltpu.sync_copy(dst_hbm.at[cid, sid], didx_v)

    @pl.when(sid < N_CH)
    def _init_chunk():
        pltpu.sync_copy(zs_hbm.at[pl.ds(sid * CH, CH)],
                        acc_sh.at[pl.ds(sid * CH, CH)])

    plsc.subcore_barrier()

    for k in range(NBUF):
        rows, sem = slots[k]
        pltpu.async_copy(zs_hbm.at[sidx_v.at[k]], rows, sem)

    def body(t, carry):
        for k in range(NBUF):
            j = NBUF * t + k
            rows, sem = slots[k]
            pltpu.make_async_copy(zs_hbm.at[sidx_v.at[j]], rows, sem).wait()
            pltpu.sync_copy(rows, acc_sh.at[didx_v.at[j]], add=True)

            @pl.when(j + NBUF < NB)
            def _prefetch():
                pltpu.async_copy(zs_hbm.at[sidx_v.at[j + NBUF]], rows, sem)
        return carry

    lax.fori_loop(0, NB // NBUF, body, 0)
    for k in range(NB - NB_MAIN):
        j = NB_MAIN + k
        rows, sem = slots[k]
        pltpu.make_async_copy(zs_hbm.at[sidx_v.at[j]], rows, sem).wait()
        pltpu.sync_copy(rows, acc_sh.at[didx_v.at[j]], add=True)
    plsc.subcore_barrier()

    @pl.when(sid < N_CH)
    def _write_chunk():
        pltpu.sync_copy(
            acc_sh.at[pl.ds(sid * CH, CH)],
            acc_out.at[cid, pl.ds(sid * CH, CH)],
        )


_agg_kernel = pl.kernel(
    _agg_body,
    out_type=jax.ShapeDtypeStruct((NC, N_NODES, HIDDEN), jnp.float32),
    mesh=_mesh,
    compiler_params=_sc_params,
    scratch_types=[
        pltpu.VMEM((NB, B), jnp.int32),
        pltpu.VMEM((NB, B), jnp.int32),
        [pltpu.VMEM((B, HIDDEN), jnp.float32) for _ in range(NBUF)],
        [pltpu.SemaphoreType.DMA for _ in range(NBUF)],
        pltpu.VMEM_SHARED((N_NODES, HIDDEN), jnp.float32),
    ],
)


BLK = 1000


def _mma_body(x_ref, h_ref, wu_ref, ww_ref, z_ref):
    z = jnp.dot(x_ref[...], wu_ref[...], preferred_element_type=jnp.float32)
    z_ref[...] = z + jnp.dot(h_ref[...], ww_ref[...],
                             preferred_element_type=jnp.float32)


_mma_kernel = pl.pallas_call(
    _mma_body,
    grid=(N_NODES // BLK,),
    in_specs=[
        pl.BlockSpec((BLK, HIDDEN), lambda i: (i, 0)),
        pl.BlockSpec((BLK, HIDDEN), lambda i: (i, 0)),
        pl.BlockSpec((HIDDEN, HIDDEN), lambda i: (0, 0)),
        pl.BlockSpec((HIDDEN, HIDDEN), lambda i: (0, 0)),
    ],
    out_specs=pl.BlockSpec((BLK, HIDDEN), lambda i: (i, 0)),
    out_shape=jax.ShapeDtypeStruct((N_NODES, HIDDEN), jnp.float32),
)


def _mmb_body(z_ref, d0_ref, d1_ref, zs_ref, dinv_ref):
    dsum = (jnp.sum(d0_ref[...], axis=1, keepdims=True)
            + jnp.sum(d1_ref[...], axis=1, keepdims=True)) * (1.0 / DEG_W) + 1.0
    dinv = lax.rsqrt(dsum)
    dinv_ref[...] = dinv
    zs_ref[...] = z_ref[...] * dinv


_mmb_kernel = pl.pallas_call(
    _mmb_body,
    grid=(N_NODES // BLK,),
    in_specs=[
        pl.BlockSpec((BLK, HIDDEN), lambda i: (i, 0)),
        pl.BlockSpec((BLK, DEG_W), lambda i: (i, 0)),
        pl.BlockSpec((BLK, DEG_W), lambda i: (i, 0)),
    ],
    out_specs=[
        pl.BlockSpec((BLK, HIDDEN), lambda i: (i, 0)),
        pl.BlockSpec((BLK, 1), lambda i: (i, 0)),
    ],
    out_shape=[
        jax.ShapeDtypeStruct((N_NODES, HIDDEN), jnp.float32),
        jax.ShapeDtypeStruct((N_NODES, 1), jnp.float32),
    ],
)


def _fin_body(a_ref, zs_ref, dinv_ref, b_ref, o_ref):
    o_ref[...] = ((a_ref[0] + a_ref[1] - zs_ref[...]) * dinv_ref[...]
                  + b_ref[...])


_fin_kernel = pl.pallas_call(
    _fin_body,
    grid=(N_NODES // BLK,),
    in_specs=[
        pl.BlockSpec((NC, BLK, HIDDEN), lambda i: (0, i, 0)),
        pl.BlockSpec((BLK, HIDDEN), lambda i: (i, 0)),
        pl.BlockSpec((BLK, 1), lambda i: (i, 0)),
        pl.BlockSpec((1, HIDDEN), lambda i: (0, 0)),
    ],
    out_specs=pl.BlockSpec((BLK, HIDDEN), lambda i: (i, 0)),
    out_shape=jax.ShapeDtypeStruct((N_NODES, HIDDEN), jnp.float32),
)


def kernel(X, H, edge_index, Wu, bu, Ww, bw):
    ei = edge_index.astype(jnp.int32)
    src = ei[0].reshape(NC, NS, NB, B)
    dst = ei[1].reshape(NC, NS, NB, B)
    z = _mma_kernel(X, H, Wu, Ww)
    ones_c = jnp.ones((B, DEG_W), jnp.float32)
    zeros_c = jnp.zeros((CH, DEG_W), jnp.float32)
    deg = _deg_kernel(dst, ones_c, zeros_c)
    zs, dinv = _mmb_kernel(z, deg[0], deg[1])
    acc = _agg_kernel(zs, src, dst)
    bias = (bu + bw).reshape(1, HIDDEN)
    return _fin_kernel(acc, zs, dinv, bias)


# speedup vs baseline: 1.0207x; 1.0207x over previous
import jax
import jax.numpy as jnp
from jax import lax
from jax.experimental import pallas as pl
from jax.experimental.pallas import tpu as pltpu
import jax.experimental.pallas.tpu_sc as plsc

N_NODES = 10000
N_EDGES = 320000
HIDDEN = 128

NC = 2
NS = 16
CH = 1000
N_CH = N_NODES // CH
DEG_W = 16

B = 40
NB = N_EDGES // NC // NS // B
NBUF = 6
NB_MAIN = (NB // NBUF) * NBUF

_mesh = plsc.VectorSubcoreMesh(core_axis_name="c", subcore_axis_name="s")
_sc_params = pltpu.CompilerParams(use_tc_tiling_on_sc=False)


FIRE = 25


def _deg_body(dst_hbm, ones_hbm, zeros_hbm, deg_out, idx_v, ones_v, sem, deg_sh):
    cid = lax.axis_index("c")
    sid = lax.axis_index("s")

    pltpu.sync_copy(ones_hbm, ones_v)
    pltpu.sync_copy(dst_hbm.at[cid, sid], idx_v)

    @pl.when(sid < N_CH)
    def _zero_chunk():
        pltpu.sync_copy(zeros_hbm, deg_sh.at[pl.ds(sid * CH, CH)])

    plsc.subcore_barrier()

    def chunk(c, carry):
        def fire(j, carry2):
            pltpu.async_copy(ones_v, deg_sh.at[idx_v.at[c * FIRE + j]], sem,
                             add=True)
            return carry2

        lax.fori_loop(0, FIRE, fire, 0)

        def drain(j, carry2):
            pltpu.make_async_copy(ones_v, deg_sh.at[idx_v.at[0]], sem).wait()
            return carry2

        lax.fori_loop(0, FIRE, drain, 0)
        return carry

    lax.fori_loop(0, NB // FIRE, chunk, 0)
    plsc.subcore_barrier()

    @pl.when(sid < N_CH)
    def _write_chunk():
        pltpu.sync_copy(
            deg_sh.at[pl.ds(sid * CH, CH)],
            deg_out.at[cid, pl.ds(sid * CH, CH)],
        )


_deg_kernel = pl.kernel(
    _deg_body,
    out_type=jax.ShapeDtypeStruct((NC, N_NODES, DEG_W), jnp.float32),
    mesh=_mesh,
    compiler_params=_sc_params,
    scratch_types=[
        pltpu.VMEM((NB, B), jnp.int32),
        pltpu.VMEM((B, DEG_W), jnp.float32),
        pltpu.SemaphoreType.DMA,
        pltpu.VMEM_SHARED((N_NODES, DEG_W), jnp.float32),
    ],
)


def _agg_body(zs_hbm, src_hbm, dst_hbm, acc_out,
              sidx_v, didx_v, rows_refs, sem_refs, acc_sh):
    cid = lax.axis_index("c")
    sid = lax.axis_index("s")
    slots = tuple(zip(rows_refs, sem_refs))

    pltpu.sync_copy(src_hbm.at[cid, sid], sidx_v)
    pltpu.sync_copy(dst_hbm.at[cid, sid], didx_v)

    @pl.when(sid < N_CH)
    def _init_chunk():
        pltpu.sync_copy(zs_hbm.at[pl.ds(sid * CH, CH)],
                        acc_sh.at[pl.ds(sid * CH, CH)])

    plsc.subcore_barrier()

    for k in range(NBUF):
        rows, sem = slots[k]
        pltpu.async_copy(zs_hbm.at[sidx_v.at[k]], rows, sem)

    def body(t, carry):
        for k in range(NBUF):
            j = NBUF * t + k
            rows, sem = slots[k]
            pltpu.make_async_copy(zs_hbm.at[sidx_v.at[j]], rows, sem).wait()
            pltpu.sync_copy(rows, acc_sh.at[didx_v.at[j]], add=True)

            @pl.when(j + NBUF < NB)
            def _prefetch():
                pltpu.async_copy(zs_hbm.at[sidx_v.at[j + NBUF]], rows, sem)
        return carry

    lax.fori_loop(0, NB // NBUF, body, 0)
    for k in range(NB - NB_MAIN):
        j = NB_MAIN + k
        rows, sem = slots[k]
        pltpu.make_async_copy(zs_hbm.at[sidx_v.at[j]], rows, sem).wait()
        pltpu.sync_copy(rows, acc_sh.at[didx_v.at[j]], add=True)
    plsc.subcore_barrier()

    @pl.when(sid < N_CH)
    def _write_chunk():
        pltpu.sync_copy(
            acc_sh.at[pl.ds(sid * CH, CH)],
            acc_out.at[cid, pl.ds(sid * CH, CH)],
        )


_agg_kernel = pl.kernel(
    _agg_body,
    out_type=jax.ShapeDtypeStruct((NC, N_NODES, HIDDEN), jnp.float32),
    mesh=_mesh,
    compiler_params=_sc_params,
    scratch_types=[
        pltpu.VMEM((NB, B), jnp.int32),
        pltpu.VMEM((NB, B), jnp.int32),
        [pltpu.VMEM((B, HIDDEN), jnp.float32) for _ in range(NBUF)],
        [pltpu.SemaphoreType.DMA for _ in range(NBUF)],
        pltpu.VMEM_SHARED((N_NODES, HIDDEN), jnp.float32),
    ],
)


BLK = 1000


def _mma_body(x_ref, h_ref, wu_ref, ww_ref, z_ref):
    z = jnp.dot(x_ref[...], wu_ref[...], preferred_element_type=jnp.float32)
    z_ref[...] = z + jnp.dot(h_ref[...], ww_ref[...],
                             preferred_element_type=jnp.float32)


_mma_kernel = pl.pallas_call(
    _mma_body,
    grid=(N_NODES // BLK,),
    in_specs=[
        pl.BlockSpec((BLK, HIDDEN), lambda i: (i, 0)),
        pl.BlockSpec((BLK, HIDDEN), lambda i: (i, 0)),
        pl.BlockSpec((HIDDEN, HIDDEN), lambda i: (0, 0)),
        pl.BlockSpec((HIDDEN, HIDDEN), lambda i: (0, 0)),
    ],
    out_specs=pl.BlockSpec((BLK, HIDDEN), lambda i: (i, 0)),
    out_shape=jax.ShapeDtypeStruct((N_NODES, HIDDEN), jnp.float32),
)


def _mmb_body(z_ref, d0_ref, d1_ref, zs_ref, dinv_ref):
    dsum = (jnp.sum(d0_ref[...], axis=1, keepdims=True)
            + jnp.sum(d1_ref[...], axis=1, keepdims=True)) * (1.0 / DEG_W) + 1.0
    dinv = lax.rsqrt(dsum)
    dinv_ref[...] = dinv
    zs_ref[...] = z_ref[...] * dinv


_mmb_kernel = pl.pallas_call(
    _mmb_body,
    grid=(N_NODES // BLK,),
    in_specs=[
        pl.BlockSpec((BLK, HIDDEN), lambda i: (i, 0)),
        pl.BlockSpec((BLK, DEG_W), lambda i: (i, 0)),
        pl.BlockSpec((BLK, DEG_W), lambda i: (i, 0)),
    ],
    out_specs=[
        pl.BlockSpec((BLK, HIDDEN), lambda i: (i, 0)),
        pl.BlockSpec((BLK, 1), lambda i: (i, 0)),
    ],
    out_shape=[
        jax.ShapeDtypeStruct((N_NODES, HIDDEN), jnp.float32),
        jax.ShapeDtypeStruct((N_NODES, 1), jnp.float32),
    ],
)


def _fin_body(a_ref, zs_ref, dinv_ref, b_ref, o_ref):
    o_ref[...] = ((a_ref[0] + a_ref[1] - zs_ref[...]) * dinv_ref[...]
                  + b_ref[...])


_fin_kernel = pl.pallas_call(
    _fin_body,
    grid=(N_NODES // BLK,),
    in_specs=[
        pl.BlockSpec((NC, BLK, HIDDEN), lambda i: (0, i, 0)),
        pl.BlockSpec((BLK, HIDDEN), lambda i: (i, 0)),
        pl.BlockSpec((BLK, 1), lambda i: (i, 0)),
        pl.BlockSpec((1, HIDDEN), lambda i: (0, 0)),
    ],
    out_specs=pl.BlockSpec((BLK, HIDDEN), lambda i: (i, 0)),
    out_shape=jax.ShapeDtypeStruct((N_NODES, HIDDEN), jnp.float32),
)


def kernel(X, H, edge_index, Wu, bu, Ww, bw):
    ei = edge_index.astype(jnp.int32)
    src = ei[0].reshape(NC, NS, NB, B)
    dst = ei[1].reshape(NC, NS, NB, B)
    z = _mma_kernel(X, H, Wu, Ww)
    ones_c = jnp.ones((B, DEG_W), jnp.float32)
    zeros_c = jnp.zeros((CH, DEG_W), jnp.float32)
    deg = _deg_kernel(dst, ones_c, zeros_c)
    zs, dinv = _mmb_kernel(z, deg[0], deg[1])
    acc = _agg_kernel(zs, src, dst)
    bias = (bu + bw).reshape(1, HIDDEN)
    return _fin_kernel(acc, zs, dinv, bias)
